# baseline (speedup 1.0000x reference)
import functools

import jax
import jax.numpy as jnp
from jax import lax
from jax.experimental import pallas as pl
from jax.experimental.pallas import tpu as pltpu
from jax.experimental.pallas import tpu_sc as plsc

_INFO = plsc.get_sparse_core_info()
_NC, _NS = _INFO.num_cores, _INFO.num_subcores
_NW = _NC * _NS


@functools.lru_cache(maxsize=None)
def _make_gather(n, v, d, chunk, nbuf):
    assert n % _NW == 0
    b_per_w = n // _NW
    assert b_per_w % chunk == 0
    n_chunks = b_per_w // chunk
    mesh = plsc.VectorSubcoreMesh(core_axis_name="c", subcore_axis_name="s")

    @functools.partial(
        pl.kernel,
        out_type=jax.ShapeDtypeStruct((n, d), jnp.float32),
        mesh=mesh,
        scratch_types=[
            pltpu.VMEM((nbuf, chunk), jnp.int32),
            pltpu.VMEM((nbuf, chunk, d), jnp.float32),
            pltpu.SemaphoreType.DMA,
            pltpu.SemaphoreType.DMA,
            pltpu.SemaphoreType.DMA,
        ],
        compiler_params=pltpu.CompilerParams(use_tc_tiling_on_sc=False),
    )
    def gather_kernel(idx_hbm, table_hbm, out_hbm, idx_v, rows_v, isem, gsem, osem):
        wid = lax.axis_index("s") * _NC + lax.axis_index("c")
        base = wid * b_per_w

        def idx_load(j):
            return pltpu.async_copy(
                idx_hbm.at[pl.ds(base + j * chunk, chunk)], idx_v.at[j % nbuf], isem)

        def gather(j):
            return pltpu.async_copy(
                table_hbm.at[idx_v.at[j % nbuf]], rows_v.at[j % nbuf], gsem)

        def store(j):
            return pltpu.async_copy(
                rows_v.at[j % nbuf], out_hbm.at[pl.ds(base + j * chunk, chunk)], osem)

        d_idx, d_g, d_o = {}, {}, {}
        d_idx[0] = idx_load(0)
        for j in range(n_chunks):
            if j + 1 < n_chunks:
                d_idx[j + 1] = idx_load(j + 1)
            d_idx[j].wait()
            if j >= nbuf:
                d_o[j - nbuf].wait()
            d_g[j] = gather(j)
            d_g[j].wait()
            d_o[j] = store(j)
        for j in range(max(0, n_chunks - nbuf), n_chunks):
            d_o[j].wait()

    return gather_kernel


def kernel(inputs, table):
    b, h = inputs.shape
    v, d = table.shape
    idx = inputs.reshape(b * h).astype(jnp.int32)
    out = _make_gather(b * h, v, d, 1600, 2)(idx, table)
    return out.reshape(b, h, d)


# speedup vs baseline: 1.6088x; 1.6088x over previous
import functools

import jax
import jax.numpy as jnp
from jax import lax
from jax.experimental import pallas as pl
from jax.experimental.pallas import tpu as pltpu
from jax.experimental.pallas import tpu_sc as plsc

_INFO = plsc.get_sparse_core_info()
_NC, _NS = _INFO.num_cores, _INFO.num_subcores
_NW = _NC * _NS


@functools.lru_cache(maxsize=None)
def _make_gather(b, h, v, d, rows_per_chunk, nbuf):
    assert b % (_NW * rows_per_chunk) == 0
    rows_per_w = b // _NW
    n_chunks = rows_per_w // rows_per_chunk
    chunk = rows_per_chunk * h
    mesh = plsc.VectorSubcoreMesh(core_axis_name="c", subcore_axis_name="s")

    @functools.partial(
        pl.kernel,
        out_type=jax.ShapeDtypeStruct((b, h, d), jnp.float32),
        mesh=mesh,
        scratch_types=[
            pltpu.VMEM((nbuf, chunk), jnp.int32),
            pltpu.VMEM((nbuf, chunk, d), jnp.float32),
            pltpu.SemaphoreType.DMA,
            pltpu.SemaphoreType.DMA,
            pltpu.SemaphoreType.DMA,
        ],
        compiler_params=pltpu.CompilerParams(use_tc_tiling_on_sc=False),
    )
    def gather_kernel(idx_hbm, table_hbm, out_hbm, idx_v, rows_v, isem, gsem, osem):
        wid = lax.axis_index("s") * _NC + lax.axis_index("c")
        row0 = wid * rows_per_w

        def idx_load(j):
            return pltpu.async_copy(
                idx_hbm.at[pl.ds((row0 + j * rows_per_chunk) * h, chunk)],
                idx_v.at[j % nbuf], isem)

        def gather(j):
            return pltpu.async_copy(
                table_hbm.at[idx_v.at[j % nbuf]], rows_v.at[j % nbuf], gsem)

        def store(j):
            s = j % nbuf
            rr = row0 + j * rows_per_chunk
            return [
                pltpu.async_copy(
                    rows_v.at[s, pl.ds(i * h, h)], out_hbm.at[rr + i], osem)
                for i in range(rows_per_chunk)
            ]

        d_idx, d_g, d_o = {}, {}, {}
        d_idx[0] = idx_load(0)
        for j in range(n_chunks):
            if j + 1 < n_chunks:
                d_idx[j + 1] = idx_load(j + 1)
            d_idx[j].wait()
            if j >= nbuf:
                for c in d_o[j - nbuf]:
                    c.wait()
            d_g[j] = gather(j)
            d_g[j].wait()
            d_o[j] = store(j)
        for j in range(max(0, n_chunks - nbuf), n_chunks):
            for c in d_o[j]:
                c.wait()

    return gather_kernel


def kernel(inputs, table):
    b, h = inputs.shape
    v, d = table.shape
    idx = inputs.reshape(b * h).astype(jnp.int32)
    return _make_gather(b, h, v, d, 32, 2)(idx, table)


# speedup vs baseline: 1.6101x; 1.0008x over previous
import functools

import jax
import jax.numpy as jnp
from jax import lax
from jax.experimental import pallas as pl
from jax.experimental.pallas import tpu as pltpu
from jax.experimental.pallas import tpu_sc as plsc

_INFO = plsc.get_sparse_core_info()
_NC, _NS = _INFO.num_cores, _INFO.num_subcores
_NW = _NC * _NS


@functools.lru_cache(maxsize=None)
def _make_gather(b, h, v, d, nbuf):
    assert b % _NW == 0
    bw = b // _NW
    mesh = plsc.VectorSubcoreMesh(core_axis_name="c", subcore_axis_name="s")

    @functools.partial(
        pl.kernel,
        out_type=jax.ShapeDtypeStruct((b, h, d), jnp.float32),
        mesh=mesh,
        scratch_types=[
            pltpu.VMEM((h, bw), jnp.int32),
            pltpu.VMEM((nbuf, bw, d), jnp.float32),
            pltpu.SemaphoreType.DMA,
            pltpu.SemaphoreType.DMA,
        ],
        compiler_params=pltpu.CompilerParams(use_tc_tiling_on_sc=False),
    )
    def gather_kernel(idx_t_hbm, table_hbm, out_hbm, idx_v, rows_v, gsem, osem):
        wid = lax.axis_index("s") * _NC + lax.axis_index("c")
        b0 = wid * bw
        pltpu.sync_copy(idx_t_hbm.at[:, pl.ds(b0, bw)], idx_v)
        d_g, d_o = {}, {}
        for j in range(h):
            s = j % nbuf
            if j >= nbuf:
                d_o[j - nbuf].wait()
            d_g[j] = pltpu.async_copy(table_hbm.at[idx_v.at[j]], rows_v.at[s], gsem)
            d_g[j].wait()
            d_o[j] = pltpu.async_copy(rows_v.at[s], out_hbm.at[pl.ds(b0, bw), j], osem)
        for j in range(max(0, h - nbuf), h):
            d_o[j].wait()

    return gather_kernel


def kernel(inputs, table):
    b, h = inputs.shape
    v, d = table.shape
    idx_t = inputs.astype(jnp.int32).T
    return _make_gather(b, h, v, d, 2)(idx_t, table)
